# baseline (speedup 1.0000x reference)
import functools

import jax
import jax.numpy as jnp
from jax import lax
from jax.experimental import pallas as pl
from jax.experimental.pallas import tpu as pltpu
from jax.experimental.pallas import tpu_sc as plsc

N_NODES = 64
N_EDGES = 64 * 64
D_MSG = 512
LANES = 16
NUM_WORKERS = 32
E_PER_W = N_EDGES // NUM_WORKERS


def _mesh():
    return plsc.VectorSubcoreMesh(core_axis_name="c", subcore_axis_name="s")


def _lane_permute(x, idx):
    dnums = lax.GatherDimensionNumbers(
        offset_dims=(), collapsed_slice_dims=(0,), start_index_map=(0,)
    )
    return lax.gather(
        x,
        idx[:, None],
        dnums,
        slice_sizes=(1,),
        mode=lax.GatherScatterMode.PROMISE_IN_BOUNDS,
    )


@functools.partial(
    pl.kernel,
    out_type=jax.ShapeDtypeStruct((NUM_WORKERS, N_NODES), jnp.float32),
    mesh=_mesh(),
    scratch_types=[
        pltpu.VMEM((E_PER_W, D_MSG), jnp.float32),
        pltpu.VMEM((E_PER_W, N_NODES), jnp.float32),
        pltpu.VMEM((1, N_NODES), jnp.float32),
    ],
)
def _partial_sums(msgs_hbm, tgts_hbm, part_hbm, m_v, t_v, acc_v):
    c = lax.axis_index("c")
    s = lax.axis_index("s")
    wid = s * 2 + c
    base = wid * E_PER_W
    pltpu.sync_copy(msgs_hbm.at[pl.ds(base, E_PER_W)], m_v)
    pltpu.sync_copy(tgts_hbm.at[pl.ds(base, E_PER_W)], t_v)

    zero = jnp.zeros((LANES,), jnp.float32)
    for j in range(N_NODES // LANES):
        acc_v[0, pl.ds(j * LANES, LANES)] = zero

    lanes = lax.iota(jnp.int32, LANES)
    perms = [lanes ^ (1 << k) for k in range(4)]

    def body(e, carry):
        svec = m_v[e, pl.ds(0, LANES)]
        for j in range(1, D_MSG // LANES):
            svec = svec + m_v[e, pl.ds(j * LANES, LANES)]
        for p in perms:
            svec = svec + _lane_permute(svec, p)
        for j in range(N_NODES // LANES):
            sl = pl.ds(j * LANES, LANES)
            acc_v[0, sl] = acc_v[0, sl] + t_v[e, sl] * svec
        return carry

    lax.fori_loop(0, E_PER_W, body, 0)
    pltpu.sync_copy(acc_v, part_hbm.at[pl.ds(wid, 1)])


@functools.partial(
    pl.kernel,
    out_type=jax.ShapeDtypeStruct((N_NODES,), jnp.float32),
    mesh=_mesh(),
    scratch_types=[
        pltpu.VMEM((NUM_WORKERS, N_NODES), jnp.float32),
        pltpu.VMEM((N_NODES,), jnp.float32),
    ],
)
def _combine(part_hbm, out_hbm, p_v, o_v):
    c = lax.axis_index("c")
    s = lax.axis_index("s")
    wid = s * 2 + c

    @pl.when(wid == 0)
    def _():
        pltpu.sync_copy(part_hbm, p_v)
        for j in range(N_NODES // LANES):
            sl = pl.ds(j * LANES, LANES)
            acc = p_v[0, sl]
            for w in range(1, NUM_WORKERS):
                acc = acc + p_v[w, sl]
            o_v[sl] = acc
        pltpu.sync_copy(o_v, out_hbm)


def kernel(edge_msgs, edge_targets):
    part = _partial_sums(edge_msgs, edge_targets)
    out = _combine(part)
    return out.reshape(N_NODES, 1)
